# baseline (speedup 1.0000x reference)
import jax
import jax.numpy as jnp
import numpy as np
from jax import lax
from jax.experimental import pallas as pl
from jax.experimental.pallas import tpu as pltpu
from jax.experimental.pallas import tpu_sc as plsc

N = 10000
E = 160000
DIN = 256
HID = 256
H = 8
HD = 32
NT = 8
ET = 16
OUT = 256
SQRT_D = float(np.sqrt(HD))
G = 2
HG = H // G
GW = HG * HD
NB = 400
NBLK = N // NB

f32 = jnp.float32


def _blockdiag_ones(rows, groups):
    per = rows // groups
    r = lax.broadcasted_iota(jnp.int32, (rows, groups), 0) // per
    c = lax.broadcasted_iota(jnp.int32, (rows, groups), 1)
    return (r == c).astype(f32)


def _tca_body(x_ref, oh_ref, w_ref, attW_ref, msgW_ref,
              kvp_ref, qg_ref, nk_ref, nq_ref):
    x = x_ref[...]
    oh = oh_ref[...]
    kqv = jnp.zeros((NB, 3 * HID), f32)
    for t in range(NT):
        xt = x * oh[:, t][:, None]
        kqv = kqv + jnp.dot(xt, w_ref[t], preferred_element_type=f32)
    k = kqv[:, :HID]
    q = kqv[:, HID:2 * HID]
    v = kqv[:, 2 * HID:]

    qg_ref[0] = q[:, :GW]
    qg_ref[1] = q[:, GW:]
    qn = jnp.dot(q * q, _blockdiag_ones(HID, H), preferred_element_type=f32)
    nq_ref[0, 0, :] = jnp.max(qn, axis=0)

    sel4 = _blockdiag_ones(GW, HG)
    nk_parts = []
    for g in range(G):
        nk_g = jnp.zeros((HG,), f32)
        for et in range(ET):
            khp = jnp.dot(k, attW_ref[g, et], preferred_element_type=f32)
            vhp = jnp.dot(v, msgW_ref[g, et], preferred_element_type=f32)
            kvp_ref[g, et] = jnp.concatenate([khp, vhp], axis=1)
            kn = jnp.dot(khp * khp, sel4, preferred_element_type=f32)
            nk_g = jnp.maximum(nk_g, jnp.max(kn, axis=0))
        nk_parts.append(nk_g)
    nk_ref[0, 0, :] = jnp.concatenate(nk_parts)


def _tc_a(x, oh, wcat, attW, msgW):
    return pl.pallas_call(
        _tca_body,
        grid=(NBLK,),
        in_specs=[
            pl.BlockSpec((NB, DIN), lambda i: (i, 0)),
            pl.BlockSpec((NB, NT), lambda i: (i, 0)),
            pl.BlockSpec((NT, DIN, 3 * HID), lambda i: (0, 0, 0)),
            pl.BlockSpec((G, ET, HID, GW), lambda i: (0, 0, 0, 0)),
            pl.BlockSpec((G, ET, HID, GW), lambda i: (0, 0, 0, 0)),
        ],
        out_specs=[
            pl.BlockSpec((G, ET, NB, 2 * GW), lambda i: (0, 0, i, 0)),
            pl.BlockSpec((G, NB, GW), lambda i: (0, i, 0)),
            pl.BlockSpec((1, 1, H), lambda i: (i, 0, 0)),
            pl.BlockSpec((1, 1, H), lambda i: (i, 0, 0)),
        ],
        out_shape=[
            jax.ShapeDtypeStruct((G, ET, N, 2 * GW), f32),
            jax.ShapeDtypeStruct((G, N, GW), f32),
            jax.ShapeDtypeStruct((NBLK, 1, H), f32),
            jax.ShapeDtypeStruct((NBLK, 1, H), f32),
        ],
    )(x, oh, wcat, attW, msgW)


def _tcb_body(acc_ref, x_ref, oh_ref, wa_ref, skip_ref, g_ref, b_ref,
              out_ref):
    x = x_ref[...]
    oh = oh_ref[...]
    expand = _blockdiag_ones(GW, HG).T
    parts = []
    for g in range(G):
        agg = acc_ref[g][:, :GW]
        s = acc_ref[g][:, GW:GW + HG] + 1e-16
        den = jnp.dot(s, expand, preferred_element_type=f32)
        parts.append(agg / den)
    a = jnp.concatenate(parts, axis=1)
    out = jnp.zeros((NB, HID), f32)
    for t in range(NT):
        at = a * oh[:, t][:, None]
        out = out + jnp.dot(at, wa_ref[t], preferred_element_type=f32)
    out = out * jax.nn.sigmoid(out)
    sg = jax.nn.sigmoid(skip_ref[...])
    a_skip = jnp.sum(oh * sg, axis=1, keepdims=True)
    out = out * a_skip + x * (1.0 - a_skip)
    mu = jnp.mean(out, axis=-1, keepdims=True)
    var = jnp.mean((out - mu) ** 2, axis=-1, keepdims=True)
    out = (out - mu) / jnp.sqrt(var + 1e-5) * g_ref[...] + b_ref[...]
    out_ref[...] = out


def _tc_b(acc, x, oh, wa, skip, gvec, bvec):
    return pl.pallas_call(
        _tcb_body,
        grid=(NBLK,),
        in_specs=[
            pl.BlockSpec((G, NB, AW), lambda i: (0, i, 0)),
            pl.BlockSpec((NB, HID), lambda i: (i, 0)),
            pl.BlockSpec((NB, NT), lambda i: (i, 0)),
            pl.BlockSpec((NT, HID, HID), lambda i: (0, 0, 0)),
            pl.BlockSpec((1, NT), lambda i: (0, 0)),
            pl.BlockSpec((1, HID), lambda i: (0, 0)),
            pl.BlockSpec((1, HID), lambda i: (0, 0)),
        ],
        out_specs=pl.BlockSpec((NB, HID), lambda i: (i, 0)),
        out_shape=jax.ShapeDtypeStruct((N, HID), f32),
    )(acc, x, oh, wa, skip, gvec, bvec)


def _tcc_body(h_ref, w_ref, b_ref, out_ref):
    out_ref[...] = (jnp.dot(h_ref[...], w_ref[...], preferred_element_type=f32)
                    + b_ref[...])


def _tc_c(h, ffW, ffb):
    return pl.pallas_call(
        _tcc_body,
        grid=(NBLK,),
        in_specs=[
            pl.BlockSpec((NB, HID), lambda i: (i, 0)),
            pl.BlockSpec((HID, OUT), lambda i: (0, 0)),
            pl.BlockSpec((1, OUT), lambda i: (0, 0)),
        ],
        out_specs=pl.BlockSpec((NB, OUT), lambda i: (i, 0)),
        out_shape=jax.ShapeDtypeStruct((N, OUT), f32),
    )(h, ffW, ffb)


EC = 32
NTILE = 16
NCHUNK = 314
EPP = NTILE * EC * NCHUNK
EPT = EPP // NTILE
NP = 10240
ROWS_PT = NP // NTILE
AW = GW + 16


def _sc_edge_body(kvp_ref, qg_ref, icat_ref, bsplat_ref, za_ref,
                  agg_out,
                  agg_sp,
                  ibuf_a, kidx_a, qidx_a, dst_a, kvr_a, qr_a, ost_a,
                  sdst_a,
                  ibuf_b, kidx_b, qidx_b, dst_b, kvr_b, qr_b, ost_b,
                  sdst_b,
                  rbuf, bv_buf, sem_a, sem_b, sem_ia, sem_ib, sem_sa,
                  sem_sb):
    g = lax.axis_index("c")
    wid = lax.axis_index("s")

    r0 = wid * ROWS_PT
    pltpu.sync_copy(za_ref.at[pl.ds(r0, ROWS_PT)],
                    agg_sp.at[pl.ds(r0, ROWS_PT)])
    pltpu.sync_copy(bsplat_ref.at[pl.ds(g * 8, 8)], bv_buf)
    plsc.subcore_barrier()

    lanes = lax.iota(jnp.int32, 16)
    base = wid * EPT
    koff = g * (ET * N)
    qoff = g * N

    A = (ibuf_a, kidx_a, qidx_a, dst_a, kvr_a, qr_a, ost_a, sem_a,
         sdst_a, sem_sa)
    B = (ibuf_b, kidx_b, qidx_b, dst_b, kvr_b, qr_b, ost_b, sem_b,
         sdst_b, sem_sb)

    def load_idx(c, S, sem_i):
        cm = lax.rem(c, NCHUNK)
        off2 = (base + cm * EC) * 2
        return pltpu.async_copy(icat_ref.at[pl.ds(off2, 2 * EC)], S[0], sem_i)

    def prep_and_fire(S):
        ibuf, kidx, qidx, dstv, kvr, qr, _, sem, _, _ = S
        for j in range(EC // 16):
            sl = pl.ds(j * 16, 16)
            e = ibuf[sl]
            d = ibuf[pl.ds(EC + j * 16, 16)]
            kidx[sl] = e + koff
            qidx[sl] = d + qoff
            dstv[sl] = d
        return [
            pltpu.async_copy(kvp_ref.at[kidx], kvr, sem),
            pltpu.async_copy(qg_ref.at[qidx], qr, sem),
        ]

    def compute_scatter(S, guard):
        _, _, _, dstv, kvr, qr, ostage, _, sdst, sem_s = S

        @pl.when(guard)
        def _wait_prev():
            pltpu.make_async_copy(ostage, agg_sp.at[sdst], sem_s).wait()

        for j2 in range(EC // 16):
            sl2 = pl.ds(j2 * 16, 16)
            sdst[sl2] = dstv[sl2]

        def edge_body(j, carry2):
            edges = (2 * j, 2 * j + 1)
            rs = []
            for i in edges:
                for hl in range(HG):
                    r = (kvr[i, pl.ds(hl * HD, 16)]
                         * qr[i, pl.ds(hl * HD, 16)]
                         + kvr[i, pl.ds(hl * HD + 16, 16)]
                         * qr[i, pl.ds(hl * HD + 16, 16)])
                    rs.append(r + lax.rev(r, (0,)))
            for kk in (8, 4, 2):
                for c in range(2 * HG):
                    rbuf[c, pl.ds(0, 16)] = rs[c]
                    rbuf[c, pl.ds(16, 16)] = rs[c]
                for c in range(2 * HG):
                    rs[c] = rs[c] + rbuf[c, pl.ds(kk, 16)]
            for sl, i in enumerate(edges):
                ev = jnp.zeros((16,), f32)
                for hl in range(HG):
                    eh = jnp.exp(rs[sl * HG + hl] - bv_buf[hl])
                    ev = jnp.where(lanes == hl, eh, ev)
                    ostage[i, pl.ds(hl * HD, 16)] = (
                        kvr[i, pl.ds(GW + hl * HD, 16)] * eh)
                    ostage[i, pl.ds(hl * HD + 16, 16)] = (
                        kvr[i, pl.ds(GW + hl * HD + 16, 16)] * eh)
                ostage[i, pl.ds(GW, 16)] = ev
            return carry2

        lax.fori_loop(0, EC // 2, edge_body, 0)
        pltpu.async_copy(ostage, agg_sp.at[sdst], sem_s, add=True)

    load_idx(0, A, sem_ia).wait()
    ga = prep_and_fire(A)
    load_idx(1, B, sem_ib)

    def pair_body(j, carry):
        c2 = 2 * j + 2
        c3 = 2 * j + 3
        load_idx(c2, A, sem_ia)
        pltpu.make_async_copy(icat_ref.at[pl.ds(0, 2 * EC)], B[0],
                              sem_ib).wait()
        gb = prep_and_fire(B)
        for cp in ga:
            cp.wait()
        compute_scatter(A, j > 0)
        pltpu.make_async_copy(icat_ref.at[pl.ds(0, 2 * EC)], A[0],
                              sem_ia).wait()
        ga2 = prep_and_fire(A)
        load_idx(c3, B, sem_ib)
        for cp in gb:
            cp.wait()
        compute_scatter(B, j > 0)
        return carry

    lax.fori_loop(0, NCHUNK // 2, pair_body, 0)
    for cp in ga:
        cp.wait()
    pltpu.make_async_copy(icat_ref.at[pl.ds(0, 2 * EC)], B[0], sem_ib).wait()
    pltpu.make_async_copy(ost_a, agg_sp.at[sdst_a], sem_sa).wait()
    pltpu.make_async_copy(ost_b, agg_sp.at[sdst_b], sem_sb).wait()

    plsc.subcore_barrier()
    pltpu.sync_copy(agg_sp.at[pl.ds(r0, ROWS_PT)],
                    agg_out.at[g, pl.ds(r0, ROWS_PT)])


def _edge_sc(kvp, qg, eidx, dst, bv):
    kvp2 = kvp.reshape(G * ET * N, 2 * GW)
    qg2 = jnp.pad(qg.reshape(G * N, GW), ((0, 8), (0, 0)))
    pad = EPP - E
    eidx = jnp.concatenate([eidx, jnp.zeros((pad,), jnp.int32)])
    dst = jnp.concatenate([dst, jnp.full((pad,), N, jnp.int32)])
    icat = jnp.stack([eidx.reshape(-1, EC), dst.reshape(-1, EC)],
                     axis=1).reshape(-1)
    bs = jnp.pad(bv, ((0, 0), (0, 8 - HG))).reshape(G * 8, 1)
    bsplat = jnp.broadcast_to(bs, (G * 8, 16))
    za = jnp.zeros((NP, AW), f32)
    mesh = plsc.VectorSubcoreMesh(core_axis_name="c", subcore_axis_name="s",
                                  num_cores=G, num_subcores=NTILE)
    bufset = [
        pltpu.VMEM((2 * EC,), jnp.int32),
        pltpu.VMEM((EC,), jnp.int32),
        pltpu.VMEM((EC,), jnp.int32),
        pltpu.VMEM((EC,), jnp.int32),
        pltpu.VMEM((EC, 2 * GW), f32),
        pltpu.VMEM((EC, GW), f32),
        pltpu.VMEM((EC, AW), f32),
        pltpu.VMEM((EC,), jnp.int32),
    ]
    fn = pl.kernel(
        _sc_edge_body,
        out_type=(
            jax.ShapeDtypeStruct((G, NP, AW), f32),
        ),
        mesh=mesh,
        scratch_types=(
            [pltpu.VMEM_SHARED((NP, AW), f32)]
            + bufset + bufset
            + [pltpu.VMEM((2 * HG, 32), f32), pltpu.VMEM((8, 16), f32)]
            + [pltpu.SemaphoreType.DMA] * 6
        ),
        compiler_params=pltpu.CompilerParams(use_tc_tiling_on_sc=False),
    )
    (out,) = fn(kvp2, qg2, icat, bsplat, za)
    return out


def _layer(x, oh, eidx, dst, wcat, attW, msgW, wa, skip, gvec, bvec):
    kvp, qg, nk, nq = _tc_a(x, oh, wcat, attW, msgW)
    nkm = jnp.sqrt(jnp.max(nk.reshape(NBLK, H), axis=0))
    nqm = jnp.sqrt(jnp.max(nq.reshape(NBLK, H), axis=0))
    bound = (nkm * nqm).reshape(G, HG)
    out = _edge_sc(kvp, qg, eidx, dst, bound)
    return _tc_b(out, x, oh, wa, skip, gvec, bvec)


def kernel(x, edge_index, ntype, etype, Wk0, Wq0, Wv0, Wa0, pri0, att0, msg0,
           skip0, g0, b0, Wk1, Wq1, Wv1, Wa1, pri1, att1, msg1, skip1, g1, b1,
           ffW, ffb):
    src = edge_index[0]
    dst = edge_index[1]
    eidx = etype * N + src
    oh = jax.nn.one_hot(ntype, NT, dtype=f32)

    def prep(Wk, Wq, Wv, att, msg, pri):
        wcat = jnp.concatenate([Wk, Wq, Wv], axis=2)
        attW = jnp.zeros((G, ET, HID, GW), f32)
        msgW = jnp.zeros((G, ET, HID, GW), f32)
        for g in range(G):
            for hl in range(HG):
                h = g * HG + hl
                scaled = att[h] * (pri[h] / SQRT_D)[:, None, None]
                attW = attW.at[g, :, h * HD:(h + 1) * HD,
                               hl * HD:(hl + 1) * HD].set(scaled)
                msgW = msgW.at[g, :, h * HD:(h + 1) * HD,
                               hl * HD:(hl + 1) * HD].set(msg[h])
        return wcat, attW, msgW

    w0 = prep(Wk0, Wq0, Wv0, att0, msg0, pri0)
    w1 = prep(Wk1, Wq1, Wv1, att1, msg1, pri1)
    h = _layer(x, oh, eidx, dst, w0[0], w0[1], w0[2],
               Wa0, skip0.reshape(1, NT), g0.reshape(1, HID),
               b0.reshape(1, HID))
    h = _layer(h, oh, eidx, dst, w1[0], w1[1], w1[2],
               Wa1, skip1.reshape(1, NT), g1.reshape(1, HID),
               b1.reshape(1, HID))
    return _tc_c(h, ffW, ffb.reshape(1, OUT))


# speedup vs baseline: 1.1711x; 1.1711x over previous
import jax
import jax.numpy as jnp
import numpy as np
from jax import lax
from jax.experimental import pallas as pl
from jax.experimental.pallas import tpu as pltpu
from jax.experimental.pallas import tpu_sc as plsc

N = 10000
E = 160000
DIN = 256
HID = 256
H = 8
HD = 32
NT = 8
ET = 16
OUT = 256
SQRT_D = float(np.sqrt(HD))
G = 2
HG = H // G
GW = HG * HD
NB = 400
NBLK = N // NB

f32 = jnp.float32


def _blockdiag_ones(rows, groups):
    per = rows // groups
    r = lax.broadcasted_iota(jnp.int32, (rows, groups), 0) // per
    c = lax.broadcasted_iota(jnp.int32, (rows, groups), 1)
    return (r == c).astype(f32)


def _tca_body(x_ref, oh_ref, w_ref, attW_ref, msgW_ref,
              kp_ref, vp_ref, qg_ref, nk_ref, nq_ref):
    x = x_ref[...]
    oh = oh_ref[...]
    kqv = jnp.zeros((NB, 3 * HID), f32)
    for t in range(NT):
        xt = x * oh[:, t][:, None]
        kqv = kqv + jnp.dot(xt, w_ref[t], preferred_element_type=f32)
    k = kqv[:, :HID]
    q = kqv[:, HID:2 * HID]
    v = kqv[:, 2 * HID:]

    qg_ref[0] = q[:, :GW]
    qg_ref[1] = q[:, GW:]
    qn = jnp.dot(q * q, _blockdiag_ones(HID, H), preferred_element_type=f32)
    nq_ref[0, 0, :] = jnp.max(qn, axis=0)

    sel4 = _blockdiag_ones(GW, HG)
    nk_parts = []
    for g in range(G):
        nk_g = jnp.zeros((HG,), f32)
        for et in range(ET):
            khp = jnp.dot(k, attW_ref[g, et], preferred_element_type=f32)
            vhp = jnp.dot(v, msgW_ref[g, et], preferred_element_type=f32)
            kp_ref[g, et] = khp
            vp_ref[g, et] = vhp
            kn = jnp.dot(khp * khp, sel4, preferred_element_type=f32)
            nk_g = jnp.maximum(nk_g, jnp.max(kn, axis=0))
        nk_parts.append(nk_g)
    nk_ref[0, 0, :] = jnp.concatenate(nk_parts)


def _tc_a(x, oh, wcat, attW, msgW):
    return pl.pallas_call(
        _tca_body,
        grid=(NBLK,),
        in_specs=[
            pl.BlockSpec((NB, DIN), lambda i: (i, 0)),
            pl.BlockSpec((NB, NT), lambda i: (i, 0)),
            pl.BlockSpec((NT, DIN, 3 * HID), lambda i: (0, 0, 0)),
            pl.BlockSpec((G, ET, HID, GW), lambda i: (0, 0, 0, 0)),
            pl.BlockSpec((G, ET, HID, GW), lambda i: (0, 0, 0, 0)),
        ],
        out_specs=[
            pl.BlockSpec((G, ET, NB, GW), lambda i: (0, 0, i, 0)),
            pl.BlockSpec((G, ET, NB, GW), lambda i: (0, 0, i, 0)),
            pl.BlockSpec((G, NB, GW), lambda i: (0, i, 0)),
            pl.BlockSpec((1, 1, H), lambda i: (i, 0, 0)),
            pl.BlockSpec((1, 1, H), lambda i: (i, 0, 0)),
        ],
        out_shape=[
            jax.ShapeDtypeStruct((G, ET, N, GW), f32),
            jax.ShapeDtypeStruct((G, ET, N, GW), f32),
            jax.ShapeDtypeStruct((G, N, GW), f32),
            jax.ShapeDtypeStruct((NBLK, 1, H), f32),
            jax.ShapeDtypeStruct((NBLK, 1, H), f32),
        ],
    )(x, oh, wcat, attW, msgW)


def _tcb_body(acc_ref, x_ref, oh_ref, wa_ref, skip_ref, g_ref, b_ref,
              out_ref):
    x = x_ref[...]
    oh = oh_ref[...]
    expand = _blockdiag_ones(GW, HG).T
    parts = []
    for g in range(G):
        agg = acc_ref[g][:, :GW]
        s = acc_ref[g][:, GW:GW + HG] + 1e-16
        den = jnp.dot(s, expand, preferred_element_type=f32)
        parts.append(agg / den)
    a = jnp.concatenate(parts, axis=1)
    out = jnp.zeros((NB, HID), f32)
    for t in range(NT):
        at = a * oh[:, t][:, None]
        out = out + jnp.dot(at, wa_ref[t], preferred_element_type=f32)
    out = out * jax.nn.sigmoid(out)
    sg = jax.nn.sigmoid(skip_ref[...])
    a_skip = jnp.sum(oh * sg, axis=1, keepdims=True)
    out = out * a_skip + x * (1.0 - a_skip)
    mu = jnp.mean(out, axis=-1, keepdims=True)
    var = jnp.mean((out - mu) ** 2, axis=-1, keepdims=True)
    out = (out - mu) / jnp.sqrt(var + 1e-5) * g_ref[...] + b_ref[...]
    out_ref[...] = out


def _tc_b(acc, x, oh, wa, skip, gvec, bvec):
    return pl.pallas_call(
        _tcb_body,
        grid=(NBLK,),
        in_specs=[
            pl.BlockSpec((G, NB, AW), lambda i: (0, i, 0)),
            pl.BlockSpec((NB, HID), lambda i: (i, 0)),
            pl.BlockSpec((NB, NT), lambda i: (i, 0)),
            pl.BlockSpec((NT, HID, HID), lambda i: (0, 0, 0)),
            pl.BlockSpec((1, NT), lambda i: (0, 0)),
            pl.BlockSpec((1, HID), lambda i: (0, 0)),
            pl.BlockSpec((1, HID), lambda i: (0, 0)),
        ],
        out_specs=pl.BlockSpec((NB, HID), lambda i: (i, 0)),
        out_shape=jax.ShapeDtypeStruct((N, HID), f32),
    )(acc, x, oh, wa, skip, gvec, bvec)


def _tcc_body(h_ref, w_ref, b_ref, out_ref):
    out_ref[...] = (jnp.dot(h_ref[...], w_ref[...], preferred_element_type=f32)
                    + b_ref[...])


def _tc_c(h, ffW, ffb):
    return pl.pallas_call(
        _tcc_body,
        grid=(NBLK,),
        in_specs=[
            pl.BlockSpec((NB, HID), lambda i: (i, 0)),
            pl.BlockSpec((HID, OUT), lambda i: (0, 0)),
            pl.BlockSpec((1, OUT), lambda i: (0, 0)),
        ],
        out_specs=pl.BlockSpec((NB, OUT), lambda i: (i, 0)),
        out_shape=jax.ShapeDtypeStruct((N, OUT), f32),
    )(h, ffW, ffb)


EC = 32
NTILE = 16
NCHUNK = 314
EPP = NTILE * EC * NCHUNK
EPT = EPP // NTILE
NP = 10240
ROWS_PT = NP // NTILE
AW = GW + 16


def _sc_edge_body(kp_ref, vp_ref, qg_ref, icat_ref, bsplat_ref, za_ref,
                  agg_out,
                  agg_sp,
                  ibuf_a, kidx_a, qidx_a, dst_a, kr_a, vr_a, qr_a, ost_a,
                  sdst_a,
                  ibuf_b, kidx_b, qidx_b, dst_b, kr_b, vr_b, qr_b, ost_b,
                  sdst_b,
                  rbuf, bv_buf, sem_a, sem_b, sem_ia, sem_ib, sem_sa,
                  sem_sb):
    g = lax.axis_index("c")
    wid = lax.axis_index("s")

    r0 = wid * ROWS_PT
    pltpu.sync_copy(za_ref.at[pl.ds(r0, ROWS_PT)],
                    agg_sp.at[pl.ds(r0, ROWS_PT)])
    pltpu.sync_copy(bsplat_ref.at[pl.ds(g * 8, 8)], bv_buf)
    plsc.subcore_barrier()

    lanes = lax.iota(jnp.int32, 16)
    base = wid * EPT
    koff = g * (ET * N)
    qoff = g * N

    A = (ibuf_a, kidx_a, qidx_a, dst_a, kr_a, vr_a, qr_a, ost_a, sem_a,
         sdst_a, sem_sa)
    B = (ibuf_b, kidx_b, qidx_b, dst_b, kr_b, vr_b, qr_b, ost_b, sem_b,
         sdst_b, sem_sb)

    def load_idx(c, S, sem_i):
        cm = lax.rem(c, NCHUNK)
        off2 = (base + cm * EC) * 2
        return pltpu.async_copy(icat_ref.at[pl.ds(off2, 2 * EC)], S[0], sem_i)

    def prep_and_fire(S):
        ibuf, kidx, qidx, dstv, kr, vr, qr, _, sem, _, _ = S
        for j in range(EC // 16):
            sl = pl.ds(j * 16, 16)
            e = ibuf[sl]
            d = ibuf[pl.ds(EC + j * 16, 16)]
            kidx[sl] = e + koff
            qidx[sl] = d + qoff
            dstv[sl] = d
        return [
            pltpu.async_copy(kp_ref.at[kidx], kr, sem),
            pltpu.async_copy(vp_ref.at[kidx], vr, sem),
            pltpu.async_copy(qg_ref.at[qidx], qr, sem),
        ]

    def compute_scatter(S, guard):
        _, _, _, dstv, kr, vr, qr, ostage, _, sdst, sem_s = S

        @pl.when(guard)
        def _wait_prev():
            pltpu.make_async_copy(ostage, agg_sp.at[sdst], sem_s).wait()

        for j2 in range(EC // 16):
            sl2 = pl.ds(j2 * 16, 16)
            sdst[sl2] = dstv[sl2]

        def edge_body(j, carry2):
            edges = (2 * j, 2 * j + 1)
            rs = []
            for i in edges:
                for hl in range(HG):
                    r = (kr[i, pl.ds(hl * HD, 16)] * qr[i, pl.ds(hl * HD, 16)]
                         + kr[i, pl.ds(hl * HD + 16, 16)]
                         * qr[i, pl.ds(hl * HD + 16, 16)])
                    rs.append(r + lax.rev(r, (0,)))
            for kk in (8, 4, 2):
                for c in range(2 * HG):
                    rbuf[c, pl.ds(0, 16)] = rs[c]
                    rbuf[c, pl.ds(16, 16)] = rs[c]
                for c in range(2 * HG):
                    rs[c] = rs[c] + rbuf[c, pl.ds(kk, 16)]
            for sl, i in enumerate(edges):
                ev = jnp.zeros((16,), f32)
                for hl in range(HG):
                    eh = jnp.exp(rs[sl * HG + hl] - bv_buf[hl])
                    ev = jnp.where(lanes == hl, eh, ev)
                    ostage[i, pl.ds(hl * HD, 16)] = (
                        vr[i, pl.ds(hl * HD, 16)] * eh)
                    ostage[i, pl.ds(hl * HD + 16, 16)] = (
                        vr[i, pl.ds(hl * HD + 16, 16)] * eh)
                ostage[i, pl.ds(GW, 16)] = ev
            return carry2

        lax.fori_loop(0, EC // 2, edge_body, 0)
        pltpu.async_copy(ostage, agg_sp.at[sdst], sem_s, add=True)

    load_idx(0, A, sem_ia).wait()
    ga = prep_and_fire(A)
    load_idx(1, B, sem_ib)

    def pair_body(j, carry):
        c2 = 2 * j + 2
        c3 = 2 * j + 3
        load_idx(c2, A, sem_ia)
        pltpu.make_async_copy(icat_ref.at[pl.ds(0, 2 * EC)], B[0],
                              sem_ib).wait()
        gb = prep_and_fire(B)
        for cp in ga:
            cp.wait()
        compute_scatter(A, j > 0)
        pltpu.make_async_copy(icat_ref.at[pl.ds(0, 2 * EC)], A[0],
                              sem_ia).wait()
        ga2 = prep_and_fire(A)
        load_idx(c3, B, sem_ib)
        for cp in gb:
            cp.wait()
        compute_scatter(B, j > 0)
        return carry

    lax.fori_loop(0, NCHUNK // 2, pair_body, 0)
    for cp in ga:
        cp.wait()
    pltpu.make_async_copy(icat_ref.at[pl.ds(0, 2 * EC)], B[0], sem_ib).wait()
    pltpu.make_async_copy(ost_a, agg_sp.at[sdst_a], sem_sa).wait()
    pltpu.make_async_copy(ost_b, agg_sp.at[sdst_b], sem_sb).wait()

    plsc.subcore_barrier()
    pltpu.sync_copy(agg_sp.at[pl.ds(r0, ROWS_PT)],
                    agg_out.at[g, pl.ds(r0, ROWS_PT)])


def _edge_sc(kp, vp, qg, eidx, dst, bv):
    kp2 = kp.reshape(G * ET * N, GW)
    vp2 = vp.reshape(G * ET * N, GW)
    qg2 = jnp.pad(qg.reshape(G * N, GW), ((0, 8), (0, 0)))
    pad = EPP - E
    eidx = jnp.concatenate([eidx, jnp.zeros((pad,), jnp.int32)])
    dst = jnp.concatenate([dst, jnp.full((pad,), N, jnp.int32)])
    icat = jnp.stack([eidx.reshape(-1, EC), dst.reshape(-1, EC)],
                     axis=1).reshape(-1)
    bs = jnp.pad(bv, ((0, 0), (0, 8 - HG))).reshape(G * 8, 1)
    bsplat = jnp.broadcast_to(bs, (G * 8, 16))
    za = jnp.zeros((NP, AW), f32)
    mesh = plsc.VectorSubcoreMesh(core_axis_name="c", subcore_axis_name="s",
                                  num_cores=G, num_subcores=NTILE)
    bufset = [
        pltpu.VMEM((2 * EC,), jnp.int32),
        pltpu.VMEM((EC,), jnp.int32),
        pltpu.VMEM((EC,), jnp.int32),
        pltpu.VMEM((EC,), jnp.int32),
        pltpu.VMEM((EC, GW), f32),
        pltpu.VMEM((EC, GW), f32),
        pltpu.VMEM((EC, GW), f32),
        pltpu.VMEM((EC, AW), f32),
        pltpu.VMEM((EC,), jnp.int32),
    ]
    fn = pl.kernel(
        _sc_edge_body,
        out_type=(
            jax.ShapeDtypeStruct((G, NP, AW), f32),
        ),
        mesh=mesh,
        scratch_types=(
            [pltpu.VMEM_SHARED((NP, AW), f32)]
            + bufset + bufset
            + [pltpu.VMEM((2 * HG, 32), f32), pltpu.VMEM((8, 16), f32)]
            + [pltpu.SemaphoreType.DMA] * 6
        ),
        compiler_params=pltpu.CompilerParams(use_tc_tiling_on_sc=False),
    )
    (out,) = fn(kp2, vp2, qg2, icat, bsplat, za)
    return out


def _layer(x, oh, eidx, dst, wcat, attW, msgW, wa, skip, gvec, bvec):
    kp, vp, qg, nk, nq = _tc_a(x, oh, wcat, attW, msgW)
    nkm = jnp.sqrt(jnp.max(nk.reshape(NBLK, H), axis=0))
    nqm = jnp.sqrt(jnp.max(nq.reshape(NBLK, H), axis=0))
    bound = (nkm * nqm).reshape(G, HG)
    out = _edge_sc(kp, vp, qg, eidx, dst, bound)
    return _tc_b(out, x, oh, wa, skip, gvec, bvec)


def kernel(x, edge_index, ntype, etype, Wk0, Wq0, Wv0, Wa0, pri0, att0, msg0,
           skip0, g0, b0, Wk1, Wq1, Wv1, Wa1, pri1, att1, msg1, skip1, g1, b1,
           ffW, ffb):
    src = edge_index[0]
    dst = edge_index[1]
    eidx = etype * N + src
    oh = jax.nn.one_hot(ntype, NT, dtype=f32)

    def prep(Wk, Wq, Wv, att, msg, pri):
        wcat = jnp.concatenate([Wk, Wq, Wv], axis=2)
        attW = jnp.zeros((G, ET, HID, GW), f32)
        msgW = jnp.zeros((G, ET, HID, GW), f32)
        for g in range(G):
            for hl in range(HG):
                h = g * HG + hl
                scaled = att[h] * (pri[h] / SQRT_D)[:, None, None]
                attW = attW.at[g, :, h * HD:(h + 1) * HD,
                               hl * HD:(hl + 1) * HD].set(scaled)
                msgW = msgW.at[g, :, h * HD:(h + 1) * HD,
                               hl * HD:(hl + 1) * HD].set(msg[h])
        return wcat, attW, msgW

    w0 = prep(Wk0, Wq0, Wv0, att0, msg0, pri0)
    w1 = prep(Wk1, Wq1, Wv1, att1, msg1, pri1)
    h = _layer(x, oh, eidx, dst, w0[0], w0[1], w0[2],
               Wa0, skip0.reshape(1, NT), g0.reshape(1, HID),
               b0.reshape(1, HID))
    h = _layer(h, oh, eidx, dst, w1[0], w1[1], w1[2],
               Wa1, skip1.reshape(1, NT), g1.reshape(1, HID),
               b1.reshape(1, HID))
    return _tc_c(h, ffW, ffb.reshape(1, OUT))
